# baseline (speedup 1.0000x reference)
import functools

import jax
import jax.numpy as jnp
from jax import lax
from jax.experimental import pallas as pl
from jax.experimental.pallas import tpu as pltpu
from jax.experimental.pallas import tpu_sc as plsc

NN = 10000
EE = 320000
DD = 128
NPAD = 10240
NC = 2
NS = 16
NW = NC * NS
EPW = EE // NW
CH = 80
NCH = EPW // CH
RPT = NPAD // NS


def _mesh():
    return plsc.VectorSubcoreMesh(core_axis_name="c", subcore_axis_name="s",
                                  num_cores=NC, num_subcores=NS)


_NBUF = 3
assert (NCH - 2) % _NBUF == 0


def _sc_agg_body(h_hbm, src_hbm, dst_hbm, zeros_hbm, out_hbm,
                 src_v, dstr_v, stage_v, zero_v, acc_sh,
                 g0, g1, g2, d0, d1, d2, s0, s1, s2, zsem):
    gsems = (g0, g1, g2)
    dsems = (d0, d1, d2)
    ssems = (s0, s1, s2)
    c = lax.axis_index("c")
    s = lax.axis_index("s")
    wid = c * NS + s
    base = s * RPT

    def start_fetch(j, b):
        pltpu.async_copy(dst_hbm.at[wid, pl.ds(j, 1)],
                         dstr_v.at[pl.ds(b, 1)], dsems[b])
        pltpu.async_copy(h_hbm.at[src_v.at[pl.ds(j * CH, CH)]],
                         stage_v.at[b], gsems[b])

    def wait_fetch(j, b):
        pltpu.make_async_copy(dst_hbm.at[wid, pl.ds(j, 1)],
                              dstr_v.at[pl.ds(b, 1)], dsems[b]).wait()
        pltpu.make_async_copy(h_hbm.at[src_v.at[pl.ds(j * CH, CH)]],
                              stage_v.at[b], gsems[b]).wait()

    def start_scatter(b):
        pltpu.async_copy(stage_v.at[b], acc_sh.at[dstr_v.at[b]], ssems[b],
                         add=True)

    def wait_scatter(b):
        pltpu.make_async_copy(stage_v.at[b], acc_sh.at[dstr_v.at[b]],
                              ssems[b]).wait()

    pltpu.sync_copy(src_hbm.at[wid], src_v)
    start_fetch(0, 0)
    start_fetch(1, 1)
    pltpu.sync_copy(zeros_hbm, zero_v)
    nz = RPT // 32
    for k in range(nz):
        pltpu.async_copy(zero_v, acc_sh.at[pl.ds(base + k * 32, 32)], zsem)
    for k in range(nz):
        pltpu.make_async_copy(zero_v, acc_sh.at[pl.ds(base + k * 32, 32)],
                              zsem).wait()
    plsc.subcore_barrier()

    def body(g, carry):
        for b in range(_NBUF):
            j = 3 * g + b
            bn = (b + 2) % _NBUF

            @pl.when(j >= 1)
            def _():
                wait_scatter(bn)
            start_fetch(j + 2, bn)
            wait_fetch(j, b)
            start_scatter(b)
        return carry

    ngrp = (NCH - 2) // _NBUF
    lax.fori_loop(0, ngrp, body, 0)
    for j in range(_NBUF * ngrp, NCH):
        b = j % _NBUF
        wait_scatter((b + 2) % _NBUF)
        wait_fetch(j, b)
        start_scatter(b)
    wait_scatter((NCH - 1) % _NBUF)
    plsc.subcore_barrier()
    pltpu.sync_copy(acc_sh.at[pl.ds(base, RPT)],
                    out_hbm.at[pl.ds(c * NPAD + base, RPT)])


def _sc_aggdeg_body(h_hbm, src_hbm, dst_hbm, zeros_hbm, zeros1_hbm,
                    out_hbm, deg_hbm,
                    src_v, dstr_v, stage_v, zero_v, acc1_v, acc_sh,
                    g0, g1, d0, d1, s0, s1, zsem):
    gsems = (g0, g1)
    dsems = (d0, d1)
    ssems = (s0, s1)
    c = lax.axis_index("c")
    s = lax.axis_index("s")
    wid = c * NS + s
    base = s * RPT

    def start_fetch(j, b):
        pltpu.async_copy(dst_hbm.at[wid, pl.ds(j, 1)],
                         dstr_v.at[pl.ds(b, 1)], dsems[b])
        pltpu.async_copy(h_hbm.at[src_v.at[pl.ds(j * CH, CH)]],
                         stage_v.at[b], gsems[b])

    def wait_fetch(j, b):
        pltpu.make_async_copy(dst_hbm.at[wid, pl.ds(j, 1)],
                              dstr_v.at[pl.ds(b, 1)], dsems[b]).wait()
        pltpu.make_async_copy(h_hbm.at[src_v.at[pl.ds(j * CH, CH)]],
                              stage_v.at[b], gsems[b]).wait()

    def start_scatter(b):
        pltpu.async_copy(stage_v.at[b], acc_sh.at[dstr_v.at[b]], ssems[b],
                         add=True)

    def wait_scatter(b):
        pltpu.make_async_copy(stage_v.at[b], acc_sh.at[dstr_v.at[b]],
                              ssems[b]).wait()

    pltpu.sync_copy(src_hbm.at[wid], src_v)
    start_fetch(0, 0)
    pltpu.sync_copy(zeros1_hbm, acc1_v)
    pltpu.sync_copy(zeros_hbm, zero_v)
    nz = RPT // 32
    for k in range(nz):
        pltpu.async_copy(zero_v, acc_sh.at[pl.ds(base + k * 32, 32)], zsem)
    for k in range(nz):
        pltpu.make_async_copy(zero_v, acc_sh.at[pl.ds(base + k * 32, 32)],
                              zsem).wait()
    plsc.subcore_barrier()
    ones = jnp.full((16,), 1.0, jnp.float32)

    def step(j, b, fetch_next, dyn_first=False):
        ob = 1 - b
        if dyn_first:
            @pl.when(j >= 1)
            def _():
                wait_scatter(ob)
        else:
            wait_scatter(ob)
        if fetch_next:
            start_fetch(j + 1, ob)
        wait_fetch(j, b)
        for k in range(CH // 16):
            idx = dstr_v[b, pl.ds(k * 16, 16)]
            plsc.addupdate_scatter(acc1_v, [idx], ones)
        start_scatter(b)

    def body(g, carry):
        step(2 * g, 0, True, dyn_first=True)
        step(2 * g + 1, 1, True)
        return carry

    lax.fori_loop(0, (NCH - 1) // 2, body, 0)
    step(NCH - 1, (NCH - 1) % 2, False)
    wait_scatter((NCH - 1) % 2)
    plsc.subcore_barrier()
    pltpu.sync_copy(acc_sh.at[pl.ds(base, RPT)],
                    out_hbm.at[pl.ds(c * NPAD + base, RPT)])
    pltpu.sync_copy(acc1_v, deg_hbm.at[wid])


@functools.lru_cache(maxsize=None)
def _sc_kernels():
    agg = pl.kernel(
        _sc_agg_body,
        mesh=_mesh(),
        out_type=jax.ShapeDtypeStruct((NC * NPAD, DD), jnp.float32),
        scratch_types=(
            [
                pltpu.VMEM((EPW,), jnp.int32),
                pltpu.VMEM((_NBUF, CH), jnp.int32),
                pltpu.VMEM((_NBUF, CH, DD), jnp.float32),
                pltpu.VMEM((32, DD), jnp.float32),
                pltpu.VMEM_SHARED((NPAD, DD), jnp.float32),
            ]
            + [pltpu.SemaphoreType.DMA] * 10
        ),
    )
    aggdeg = pl.kernel(
        _sc_aggdeg_body,
        mesh=_mesh(),
        compiler_params=pltpu.CompilerParams(needs_layout_passes=False),
        out_type=(jax.ShapeDtypeStruct((NC * NPAD, DD), jnp.float32),
                  jax.ShapeDtypeStruct((NW, NPAD), jnp.float32)),
        scratch_types=(
            [
                pltpu.VMEM((EPW,), jnp.int32),
                pltpu.VMEM((2, CH), jnp.int32),
                pltpu.VMEM((2, CH, DD), jnp.float32),
                pltpu.VMEM((32, DD), jnp.float32),
                pltpu.VMEM((NPAD,), jnp.float32),
                pltpu.VMEM_SHARED((NPAD, DD), jnp.float32),
            ]
            + [pltpu.SemaphoreType.DMA] * 7
        ),
    )
    return agg, aggdeg


def _tc_body(relu):
    def f(p0, p1, d, h, wl, blp, wr, o):
        deg = jnp.maximum(jnp.sum(d[...], axis=0), 1.0)[:, None]
        agg = (p0[...] + p1[...]) / deg
        y = lax.dot_general(agg, wl[...], (((1,), (1,)), ((), ())),
                            preferred_element_type=jnp.float32)
        y = y + blp[...][0:1, :]
        y = y + lax.dot_general(h[...], wr[...], (((1,), (1,)), ((), ())),
                                preferred_element_type=jnp.float32)
        if relu:
            y = jnp.maximum(y, 0.0)
        o[...] = y
    return f


_BM = 1024
_NBLK = NPAD // _BM


def _dense(parts, deg, h, wl, bl, wr, relu):
    blp = jnp.broadcast_to(bl[None, :], (8, DD))
    return pl.pallas_call(
        _tc_body(relu),
        grid=(_NBLK,),
        in_specs=[
            pl.BlockSpec((_BM, DD), lambda i: (i, 0)),
            pl.BlockSpec((_BM, DD), lambda i: (_NBLK + i, 0)),
            pl.BlockSpec((NW, _BM), lambda i: (0, i)),
            pl.BlockSpec((_BM, DD), lambda i: (i, 0)),
            pl.BlockSpec((DD, DD), lambda i: (0, 0)),
            pl.BlockSpec((8, DD), lambda i: (0, 0)),
            pl.BlockSpec((DD, DD), lambda i: (0, 0)),
        ],
        out_specs=pl.BlockSpec((_BM, DD), lambda i: (i, 0)),
        out_shape=jax.ShapeDtypeStruct((NPAD, DD), jnp.float32),
    )(parts, parts, deg, h, wl, blp, wr)


def kernel(x, edge_index, Wl1, bl1, Wr1, Wl2, bl2, Wr2, Wl3, bl3, Wr3):
    src = edge_index[0].reshape(NW, EPW)
    dst = edge_index[1].reshape(NW, NCH, CH)
    xp = jnp.pad(x, ((0, NPAD - NN), (0, 0)))
    zeros_d = jnp.zeros((32, DD), jnp.float32)
    zeros_1 = jnp.zeros((NPAD,), jnp.float32)

    sc_agg, sc_aggdeg = _sc_kernels()
    parts, deg = sc_aggdeg(xp, src, dst, zeros_d, zeros_1)
    h = _dense(parts, deg, xp, Wl1, bl1, Wr1, True)
    for wl, bl, wr, relu in ((Wl2, bl2, Wr2, False),
                             (Wl3, bl3, Wr3, False)):
        parts = sc_agg(h, src, dst, zeros_d)
        h = _dense(parts, deg, h, wl, bl, wr, relu)
    return h[:NN]


# speedup vs baseline: 1.0354x; 1.0354x over previous
import functools

import jax
import jax.numpy as jnp
from jax import lax
from jax.experimental import pallas as pl
from jax.experimental.pallas import tpu as pltpu
from jax.experimental.pallas import tpu_sc as plsc

NN = 10000
EE = 320000
DD = 128
NPAD = 10240
NC = 2
NS = 16
NW = NC * NS
EPW = EE // NW
CH = 80
NCH = EPW // CH
RPT = NPAD // NS


def _mesh():
    return plsc.VectorSubcoreMesh(core_axis_name="c", subcore_axis_name="s",
                                  num_cores=NC, num_subcores=NS)


_NBUF = 3
assert (NCH - 2) % _NBUF == 0


def _sc_agg_body(h_hbm, src_hbm, dst_hbm, zeros_hbm, out_hbm,
                 src_v, dstr_v, stage_v, zero_v, acc_sh,
                 g0, g1, g2, d0, d1, d2, s0, s1, s2, zsem):
    gsems = (g0, g1, g2)
    dsems = (d0, d1, d2)
    ssems = (s0, s1, s2)
    c = lax.axis_index("c")
    s = lax.axis_index("s")
    wid = c * NS + s
    base = s * RPT

    def start_fetch(j, b):
        pltpu.async_copy(dst_hbm.at[wid, pl.ds(j, 1)],
                         dstr_v.at[pl.ds(b, 1)], dsems[b])
        pltpu.async_copy(h_hbm.at[src_v.at[pl.ds(j * CH, CH)]],
                         stage_v.at[b], gsems[b])

    def wait_fetch(j, b):
        pltpu.make_async_copy(dst_hbm.at[wid, pl.ds(j, 1)],
                              dstr_v.at[pl.ds(b, 1)], dsems[b]).wait()
        pltpu.make_async_copy(h_hbm.at[src_v.at[pl.ds(j * CH, CH)]],
                              stage_v.at[b], gsems[b]).wait()

    def start_scatter(b):
        pltpu.async_copy(stage_v.at[b], acc_sh.at[dstr_v.at[b]], ssems[b],
                         add=True)

    def wait_scatter(b):
        pltpu.make_async_copy(stage_v.at[b], acc_sh.at[dstr_v.at[b]],
                              ssems[b]).wait()

    pltpu.sync_copy(src_hbm.at[wid], src_v)
    start_fetch(0, 0)
    start_fetch(1, 1)
    pltpu.sync_copy(zeros_hbm, zero_v)
    nz = RPT // 32
    for k in range(nz):
        pltpu.async_copy(zero_v, acc_sh.at[pl.ds(base + k * 32, 32)], zsem)
    for k in range(nz):
        pltpu.make_async_copy(zero_v, acc_sh.at[pl.ds(base + k * 32, 32)],
                              zsem).wait()
    plsc.subcore_barrier()

    def body(g, carry):
        for b in range(_NBUF):
            j = 3 * g + b
            bn = (b + 2) % _NBUF

            @pl.when(j >= 1)
            def _():
                wait_scatter(bn)
            start_fetch(j + 2, bn)
            wait_fetch(j, b)
            start_scatter(b)
        return carry

    ngrp = (NCH - 2) // _NBUF
    lax.fori_loop(0, ngrp, body, 0)
    for j in range(_NBUF * ngrp, NCH):
        b = j % _NBUF
        wait_scatter((b + 2) % _NBUF)
        wait_fetch(j, b)
        start_scatter(b)
    wait_scatter((NCH - 1) % _NBUF)
    plsc.subcore_barrier()
    pltpu.sync_copy(acc_sh.at[pl.ds(base, RPT)],
                    out_hbm.at[pl.ds(c * NPAD + base, RPT)])


def _sc_deg_body(dst_hbm, zeros_hbm, out_hbm, dst_v, acc_v):
    c = lax.axis_index("c")
    s = lax.axis_index("s")
    wid = c * NS + s
    pltpu.sync_copy(zeros_hbm, acc_v)
    pltpu.sync_copy(dst_hbm.at[wid], dst_v)
    ones = jnp.full((16,), 1.0, jnp.float32)

    def body(i, carry):
        idx = dst_v[pl.ds(i * 16, 16)]
        plsc.addupdate_scatter(acc_v, [idx], ones)
        return carry

    lax.fori_loop(0, EPW // 16, body, 0)
    pltpu.sync_copy(acc_v, out_hbm.at[wid])


@functools.lru_cache(maxsize=None)
def _sc_kernels():
    agg = pl.kernel(
        _sc_agg_body,
        mesh=_mesh(),
        out_type=jax.ShapeDtypeStruct((NC * NPAD, DD), jnp.float32),
        scratch_types=(
            [
                pltpu.VMEM((EPW,), jnp.int32),
                pltpu.VMEM((_NBUF, CH), jnp.int32),
                pltpu.VMEM((_NBUF, CH, DD), jnp.float32),
                pltpu.VMEM((32, DD), jnp.float32),
                pltpu.VMEM_SHARED((NPAD, DD), jnp.float32),
            ]
            + [pltpu.SemaphoreType.DMA] * 10
        ),
    )
    deg = pl.kernel(
        _sc_deg_body,
        mesh=_mesh(),
        compiler_params=pltpu.CompilerParams(needs_layout_passes=False),
        out_type=jax.ShapeDtypeStruct((NW, NPAD), jnp.float32),
        scratch_types=[
            pltpu.VMEM((EPW,), jnp.int32),
            pltpu.VMEM((NPAD,), jnp.float32),
        ],
    )
    return agg, deg


def _tc_body(relu):
    def f(p0, p1, d, h, wl, blp, wr, o):
        deg = jnp.maximum(jnp.sum(d[...], axis=0), 1.0)[:, None]
        agg = (p0[...] + p1[...]) / deg
        y = lax.dot_general(agg, wl[...], (((1,), (1,)), ((), ())),
                            preferred_element_type=jnp.float32)
        y = y + blp[...][0:1, :]
        y = y + lax.dot_general(h[...], wr[...], (((1,), (1,)), ((), ())),
                                preferred_element_type=jnp.float32)
        if relu:
            y = jnp.maximum(y, 0.0)
        o[...] = y
    return f


_BM = 1024
_NBLK = NPAD // _BM


def _dense(parts, deg, h, wl, bl, wr, relu):
    blp = jnp.broadcast_to(bl[None, :], (8, DD))
    return pl.pallas_call(
        _tc_body(relu),
        grid=(_NBLK,),
        in_specs=[
            pl.BlockSpec((_BM, DD), lambda i: (i, 0)),
            pl.BlockSpec((_BM, DD), lambda i: (_NBLK + i, 0)),
            pl.BlockSpec((NW, _BM), lambda i: (0, i)),
            pl.BlockSpec((_BM, DD), lambda i: (i, 0)),
            pl.BlockSpec((DD, DD), lambda i: (0, 0)),
            pl.BlockSpec((8, DD), lambda i: (0, 0)),
            pl.BlockSpec((DD, DD), lambda i: (0, 0)),
        ],
        out_specs=pl.BlockSpec((_BM, DD), lambda i: (i, 0)),
        out_shape=jax.ShapeDtypeStruct((NPAD, DD), jnp.float32),
    )(parts, parts, deg, h, wl, blp, wr)


def kernel(x, edge_index, Wl1, bl1, Wr1, Wl2, bl2, Wr2, Wl3, bl3, Wr3):
    src = edge_index[0].reshape(NW, EPW)
    dst = edge_index[1].reshape(NW, NCH, CH)
    xp = jnp.pad(x, ((0, NPAD - NN), (0, 0)))
    zeros_d = jnp.zeros((32, DD), jnp.float32)
    zeros_1 = jnp.zeros((NPAD,), jnp.float32)

    sc_agg, sc_deg = _sc_kernels()
    deg = sc_deg(edge_index[1].reshape(NW, EPW), zeros_1)
    h = xp
    for wl, bl, wr, relu in ((Wl1, bl1, Wr1, True),
                             (Wl2, bl2, Wr2, False),
                             (Wl3, bl3, Wr3, False)):
        parts = sc_agg(h, src, dst, zeros_d)
        h = _dense(parts, deg, h, wl, bl, wr, relu)
    return h[:NN]


# speedup vs baseline: 1.0374x; 1.0019x over previous
import functools

import jax
import jax.numpy as jnp
from jax import lax
from jax.experimental import pallas as pl
from jax.experimental.pallas import tpu as pltpu
from jax.experimental.pallas import tpu_sc as plsc

NN = 10000
EE = 320000
DD = 128
NPAD = 10240
NC = 2
NS = 16
NW = NC * NS
EPW = EE // NW
CH = 80
NCH = EPW // CH
RPT = NPAD // NS


def _mesh():
    return plsc.VectorSubcoreMesh(core_axis_name="c", subcore_axis_name="s",
                                  num_cores=NC, num_subcores=NS)


_NBUF = 3
assert (NCH - 2) % _NBUF == 0


def _sc_agg_body(h_hbm, src_hbm, dst_hbm, zeros_hbm, out_hbm,
                 src_v, dstr_v, stage_v, zero_v, acc_sh,
                 g0, g1, g2, d0, d1, d2, s0, s1, s2, zsem):
    gsems = (g0, g1, g2)
    dsems = (d0, d1, d2)
    ssems = (s0, s1, s2)
    c = lax.axis_index("c")
    s = lax.axis_index("s")
    wid = c * NS + s
    base = s * RPT

    def start_fetch(j, b):
        pltpu.async_copy(dst_hbm.at[wid, pl.ds(j, 1)],
                         dstr_v.at[pl.ds(b, 1)], dsems[b])
        pltpu.async_copy(h_hbm.at[src_v.at[pl.ds(j * CH, CH)]],
                         stage_v.at[b], gsems[b])

    def wait_fetch(j, b):
        pltpu.make_async_copy(dst_hbm.at[wid, pl.ds(j, 1)],
                              dstr_v.at[pl.ds(b, 1)], dsems[b]).wait()
        pltpu.make_async_copy(h_hbm.at[src_v.at[pl.ds(j * CH, CH)]],
                              stage_v.at[b], gsems[b]).wait()

    def start_scatter(b):
        pltpu.async_copy(stage_v.at[b], acc_sh.at[dstr_v.at[b]], ssems[b],
                         add=True)

    def wait_scatter(b):
        pltpu.make_async_copy(stage_v.at[b], acc_sh.at[dstr_v.at[b]],
                              ssems[b]).wait()

    pltpu.sync_copy(src_hbm.at[wid], src_v)
    start_fetch(0, 0)
    start_fetch(1, 1)
    pltpu.sync_copy(zeros_hbm, zero_v)
    nz = RPT // 40
    for k in range(nz):
        pltpu.async_copy(zero_v, acc_sh.at[pl.ds(base + k * 40, 40)], zsem)
    for k in range(nz):
        pltpu.make_async_copy(zero_v, acc_sh.at[pl.ds(base + k * 40, 40)],
                              zsem).wait()
    plsc.subcore_barrier()

    def body(g, carry):
        for b in range(_NBUF):
            j = 3 * g + b
            bn = (b + 2) % _NBUF

            @pl.when(j >= 1)
            def _():
                wait_scatter(bn)
            start_fetch(j + 2, bn)
            wait_fetch(j, b)
            start_scatter(b)
        return carry

    ngrp = (NCH - 2) // _NBUF
    lax.fori_loop(0, ngrp, body, 0)
    for j in range(_NBUF * ngrp, NCH):
        b = j % _NBUF
        wait_scatter((b + 2) % _NBUF)
        wait_fetch(j, b)
        start_scatter(b)
    wait_scatter((NCH - 1) % _NBUF)
    plsc.subcore_barrier()
    pltpu.sync_copy(acc_sh.at[pl.ds(base, RPT)],
                    out_hbm.at[pl.ds(c * NPAD + base, RPT)])


def _sc_deg_body(dst_hbm, zeros_hbm, out_hbm, dst_v, acc_v):
    c = lax.axis_index("c")
    s = lax.axis_index("s")
    wid = c * NS + s
    pltpu.sync_copy(zeros_hbm, acc_v)
    pltpu.sync_copy(dst_hbm.at[wid], dst_v)
    ones = jnp.full((16,), 1.0, jnp.float32)

    def body(i, carry):
        for u in range(5):
            idx = dst_v[pl.ds((5 * i + u) * 16, 16)]
            plsc.addupdate_scatter(acc_v, [idx], ones)
        return carry

    lax.fori_loop(0, EPW // 80, body, 0)
    pltpu.sync_copy(acc_v, out_hbm.at[wid])


@functools.lru_cache(maxsize=None)
def _sc_kernels():
    agg = pl.kernel(
        _sc_agg_body,
        mesh=_mesh(),
        out_type=jax.ShapeDtypeStruct((NC * NPAD, DD), jnp.float32),
        scratch_types=(
            [
                pltpu.VMEM((EPW,), jnp.int32),
                pltpu.VMEM((_NBUF, CH), jnp.int32),
                pltpu.VMEM((_NBUF, CH, DD), jnp.float32),
                pltpu.VMEM((40, DD), jnp.float32),
                pltpu.VMEM_SHARED((NPAD, DD), jnp.float32),
            ]
            + [pltpu.SemaphoreType.DMA] * 10
        ),
    )
    deg = pl.kernel(
        _sc_deg_body,
        mesh=_mesh(),
        compiler_params=pltpu.CompilerParams(needs_layout_passes=False),
        out_type=jax.ShapeDtypeStruct((NW, NPAD), jnp.float32),
        scratch_types=[
            pltpu.VMEM((EPW,), jnp.int32),
            pltpu.VMEM((NPAD,), jnp.float32),
        ],
    )
    return agg, deg


def _tc_body(relu):
    def f(p0, p1, d, h, wl, blp, wr, o):
        deg = jnp.maximum(jnp.sum(d[...], axis=0), 1.0)[:, None]
        agg = (p0[...] + p1[...]) / deg
        y = lax.dot_general(agg, wl[...], (((1,), (1,)), ((), ())),
                            preferred_element_type=jnp.float32)
        y = y + blp[...][0:1, :]
        y = y + lax.dot_general(h[...], wr[...], (((1,), (1,)), ((), ())),
                                preferred_element_type=jnp.float32)
        if relu:
            y = jnp.maximum(y, 0.0)
        o[...] = y
    return f


_BM = 1024
_NBLK = NPAD // _BM


def _dense(parts, deg, h, wl, bl, wr, relu):
    blp = jnp.broadcast_to(bl[None, :], (8, DD))
    return pl.pallas_call(
        _tc_body(relu),
        grid=(_NBLK,),
        in_specs=[
            pl.BlockSpec((_BM, DD), lambda i: (i, 0)),
            pl.BlockSpec((_BM, DD), lambda i: (_NBLK + i, 0)),
            pl.BlockSpec((NW, _BM), lambda i: (0, i)),
            pl.BlockSpec((_BM, DD), lambda i: (i, 0)),
            pl.BlockSpec((DD, DD), lambda i: (0, 0)),
            pl.BlockSpec((8, DD), lambda i: (0, 0)),
            pl.BlockSpec((DD, DD), lambda i: (0, 0)),
        ],
        out_specs=pl.BlockSpec((_BM, DD), lambda i: (i, 0)),
        out_shape=jax.ShapeDtypeStruct((NPAD, DD), jnp.float32),
    )(parts, parts, deg, h, wl, blp, wr)


def kernel(x, edge_index, Wl1, bl1, Wr1, Wl2, bl2, Wr2, Wl3, bl3, Wr3):
    src = edge_index[0].reshape(NW, EPW)
    dst = edge_index[1].reshape(NW, NCH, CH)
    xp = jnp.pad(x, ((0, NPAD - NN), (0, 0)))
    zeros_d = jnp.zeros((40, DD), jnp.float32)
    zeros_1 = jnp.zeros((NPAD,), jnp.float32)

    sc_agg, sc_deg = _sc_kernels()
    deg = sc_deg(edge_index[1].reshape(NW, EPW), zeros_1)
    h = xp
    for wl, bl, wr, relu in ((Wl1, bl1, Wr1, True),
                             (Wl2, bl2, Wr2, False),
                             (Wl3, bl3, Wr3, False)):
        parts = sc_agg(h, src, dst, zeros_d)
        h = _dense(parts, deg, h, wl, bl, wr, relu)
    return h[:NN]
